# baseline (speedup 1.0000x reference)
import functools

import jax
import jax.numpy as jnp
from jax import lax
from jax.experimental import pallas as pl
from jax.experimental.pallas import tpu as pltpu
from jax.experimental.pallas import tpu_sc as plsc

_NEG = float("-inf")

_N = 100000
_HALF = 50000
_L = 16
_CH = 128
_NSEG = 512
_RG = 8
_CG = 4
_TR = 520


def _k1_body(x_ref, w1_ref, b1_ref, w2_ref, batch_ref, lg_ref, m_ref,
             rmax_ref, *, nseg, blk):
    i = pl.program_id(0)
    nb = pl.num_programs(0)

    @pl.when(i == 0)
    def _init():
        rmax_ref[...] = jnp.full((nseg, 1), _NEG, jnp.bfloat16)

    x = x_ref[...]
    h = jnp.dot(x, w1_ref[...], preferred_element_type=jnp.float32)
    h = h + b1_ref[...]
    h = h * jax.nn.sigmoid(h)
    lt_row = lax.dot_general(w2_ref[...], h, (((0,), (1,)), ((), ())),
                             preferred_element_type=jnp.float32)
    lg_ref[...] = lt_row.reshape(1, 1, blk)

    bt = batch_ref[0].astype(jnp.int16)
    seg = lax.broadcasted_iota(jnp.int16, (nseg, 1), 0)
    oh = bt == seg
    ltb = lt_row.astype(jnp.bfloat16)
    bmax = jnp.max(jnp.where(oh, ltb, jnp.bfloat16(_NEG)),
                   axis=1, keepdims=True)
    nm = jnp.maximum(rmax_ref[...], bmax)
    rmax_ref[...] = nm

    @pl.when(i == nb - 1)
    def _fin():
        m_ref[...] = nm.astype(jnp.float32)


def _k1(x, W1, b1r, w2r, batch3, nseg, blk, nbh, d, h, off):
    return pl.pallas_call(
        functools.partial(_k1_body, nseg=nseg, blk=blk),
        grid=(nbh,),
        in_specs=[
            pl.BlockSpec((blk, d), lambda i: (i + off, 0)),
            pl.BlockSpec((d, h), lambda i: (0, 0)),
            pl.BlockSpec((1, h), lambda i: (0, 0)),
            pl.BlockSpec((h, 1), lambda i: (0, 0)),
            pl.BlockSpec((1, 1, blk), lambda i: (i + off, 0, 0)),
        ],
        out_specs=[
            pl.BlockSpec((1, 1, blk), lambda i: (i, 0, 0)),
            pl.BlockSpec((nseg, 1), lambda i: (0, 0)),
        ],
        out_shape=[
            jax.ShapeDtypeStruct((nbh, 1, blk), jnp.float32),
            jax.ShapeDtypeStruct((nseg, 1), jnp.float32),
        ],
        scratch_shapes=[pltpu.VMEM((nseg, 1), jnp.bfloat16)],
    )(x, W1, b1r, w2r, batch3)


def _k2_body(x_hbm, lg_hbm, b_hbm, m_hbm, feat_hbm, den_hbm,
             xv, lv, bv, mv, tab, dtab, sems, *, lo):
    c = lax.axis_index("c")
    s = lax.axis_index("s")
    wid = c * 16 + s
    rw = wid // _CG
    cg = wid % _CG
    c0 = 48 * rw + jnp.minimum(rw, 6)
    cnt = 48 + (rw < 6).astype(jnp.int32) + (rw == 7).astype(jnp.int32)

    pltpu.sync_copy(m_hbm, mv)

    zero = jnp.zeros((_L,), jnp.float32)

    def zrow(i, carry):
        for k in range(128 // _L):
            tab[i, pl.ds(k * _L, _L)] = zero
        return carry

    lax.fori_loop(0, _TR, zrow, 0)

    def zdrow(i, carry):
        for k in range(128 // _L):
            dtab[i, pl.ds(k * _L, _L)] = zero
        return carry

    lax.fori_loop(0, 64, zdrow, 0)

    iota = lax.iota(jnp.int32, _L)
    lane0 = (iota == 0).astype(jnp.float32)
    col_base = cg * 128

    def chunk_base(ci):
        return jnp.minimum(ci * _CH, _HALF - _CH)

    def fire(ci, b):
        base = chunk_base(ci)
        pltpu.async_copy(
            x_hbm.at[pl.ds(lo + base, _CH), pl.ds(col_base, 128)], xv.at[b],
            sems.at[b])
        pltpu.async_copy(lg_hbm.at[pl.ds(base, _CH)], lv.at[b], sems.at[b])
        pltpu.async_copy(b_hbm.at[pl.ds(base, _CH)], bv.at[b], sems.at[b])

    def drain(ci, b):
        base = chunk_base(ci)
        pltpu.make_async_copy(
            x_hbm.at[pl.ds(lo + base, _CH), pl.ds(col_base, 128)], xv.at[b],
            sems.at[b]).wait()
        pltpu.make_async_copy(
            lg_hbm.at[pl.ds(base, _CH)], lv.at[b], sems.at[b]).wait()
        pltpu.make_async_copy(
            b_hbm.at[pl.ds(base, _CH)], bv.at[b], sems.at[b]).wait()

    def flush(tgt, acc, accd):
        for k in range(128 // _L):
            tab[tgt, pl.ds(k * _L, _L)] = tab[tgt, pl.ds(k * _L, _L)] + acc[k]
        plsc.addupdate_scatter(
            dtab,
            [jnp.full((_L,), lax.shift_right_logical(tgt, 3), jnp.int32),
             jnp.full((_L,), lax.bitwise_and(tgt, 7) * _L, jnp.int32) + iota],
            accd)

    fire(c0, 0)

    def chunk_body(i, carry):
        ci = c0 + i
        b = lax.rem(i, 2)

        @pl.when(i + 1 < cnt)
        def _():
            fire(ci + 1, 1 - b)

        drain(ci, b)
        glo = jnp.where(ci * _CH > _HALF - _CH, (_CH - 80) // _L, 0)

        def group_body(g, gc):
            acc = gc[:8]
            accd = gc[8]
            cur = gc[9]
            b16 = bv[b, pl.ds(g * _L, _L)]
            l16 = lv[b, pl.ds(g * _L, _L)]
            m16 = plsc.load_gather(mv, [b16])
            e16 = jnp.exp(l16 - m16)
            seg0 = b16[0]

            def fast(*op):
                facc = list(op[:8])
                faccd = op[8]
                fcur = op[9]

                @pl.when(seg0 != fcur)
                def _():
                    flush(jnp.maximum(fcur, 0), facc, faccd)

                keep = jnp.where(seg0 == fcur, 1.0, 0.0)
                facc = [a * keep for a in facc]
                faccd = faccd * keep + e16
                for r in range(_L):
                    e_b = e16[r]
                    for k in range(128 // _L):
                        facc[k] = facc[k] + e_b * xv[
                            b, g * _L + r, pl.ds(k * _L, _L)]
                return tuple(facc) + (faccd, seg0)

            def slow(*op):
                sacc = list(op[:8])
                saccd = op[8]
                scur = op[9]
                for r in range(_L):
                    seg = b16[r]
                    e_b = e16[r]

                    @pl.when(seg != scur)
                    def _(sacc=sacc, saccd=saccd, scur=scur):
                        flush(jnp.maximum(scur, 0), sacc, saccd)

                    keep = jnp.where(seg == scur, 1.0, 0.0)
                    for k in range(128 // _L):
                        sacc[k] = sacc[k] * keep + e_b * xv[
                            b, g * _L + r, pl.ds(k * _L, _L)]
                    saccd = saccd * keep + e_b * lane0
                    scur = seg
                return tuple(sacc) + (saccd, scur)

            return lax.cond(seg0 == b16[_L - 1], fast, slow, *gc)

        return lax.fori_loop(glo, _CH // _L, group_body, carry)

    carry0 = tuple(zero for _ in range(9)) + (jnp.int32(-1),)
    fc = lax.fori_loop(0, cnt, chunk_body, carry0)
    flush(jnp.maximum(fc[9], 0), list(fc[:8]), fc[8])

    pltpu.sync_copy(tab, feat_hbm.at[wid])
    pltpu.sync_copy(dtab, den_hbm.at[wid])


def _k2(x, lg, batch, m, lo):
    mesh = plsc.VectorSubcoreMesh(core_axis_name="c", subcore_axis_name="s")
    f = pl.kernel(
        functools.partial(_k2_body, lo=lo),
        out_type=[
            jax.ShapeDtypeStruct((_RG * _CG, _TR, 128), jnp.float32),
            jax.ShapeDtypeStruct((_RG * _CG, 64, 128), jnp.float32),
        ],
        mesh=mesh,
        compiler_params=pltpu.CompilerParams(needs_layout_passes=False),
        scratch_types=[
            pltpu.VMEM((2, _CH, 128), jnp.float32),
            pltpu.VMEM((2, _CH), jnp.float32),
            pltpu.VMEM((2, _CH), jnp.int32),
            pltpu.VMEM((_NSEG,), jnp.float32),
            pltpu.VMEM((_TR, 128), jnp.float32),
            pltpu.VMEM((64, 128), jnp.float32),
            pltpu.SemaphoreType.DMA((2,)),
        ],
    )
    return f(x, lg, batch, m)


def _den_fold(d):
    dsum = jnp.sum(d, axis=0)
    srow = lax.broadcasted_iota(jnp.int32, (_NSEG, 64), 0)
    rcol = lax.broadcasted_iota(jnp.int32, (_NSEG, 64), 1)
    sel = (rcol == lax.shift_right_logical(srow, 3)).astype(jnp.float32)
    g = jnp.dot(sel, dsum, preferred_element_type=jnp.float32)
    sc = lax.broadcasted_iota(jnp.int32, (_NSEG, 128), 0)
    cc = lax.broadcasted_iota(jnp.int32, (_NSEG, 128), 1)
    win = (lax.shift_right_logical(cc, 4) ==
           lax.bitwise_and(sc, 7)).astype(jnp.float32)
    return jnp.sum(g * win, axis=1, keepdims=True) * 0.25


def _k3_body(pa_ref, pb_ref, da_ref, db_ref, ma_ref, mb_ref, out_ref):
    ma = ma_ref[...]
    mb = mb_ref[...]
    m = jnp.maximum(ma, mb)
    sa = jnp.where(ma == _NEG, 0.0, jnp.exp(ma - m))
    sb = jnp.where(mb == _NEG, 0.0, jnp.exp(mb - m))
    pa = pa_ref[...][:, :, :_NSEG, :]
    pb = pb_ref[...][:, :, :_NSEG, :]
    fa = jnp.sum(pa, axis=0)
    fb = jnp.sum(pb, axis=0)
    feat = jnp.concatenate(
        [fa[g] * sa + fb[g] * sb for g in range(_CG)], axis=1)
    den = _den_fold(da_ref[...]) * sa + _den_fold(db_ref[...]) * sb
    out_ref[...] = feat / (den + 1e-16)


def _k3(pa, pb, da, db, ma, mb):
    return pl.pallas_call(
        _k3_body,
        in_specs=[
            pl.BlockSpec((_RG, _CG, _TR, 128), lambda: (0, 0, 0, 0)),
            pl.BlockSpec((_RG, _CG, _TR, 128), lambda: (0, 0, 0, 0)),
            pl.BlockSpec((_RG * _CG, 64, 128), lambda: (0, 0, 0)),
            pl.BlockSpec((_RG * _CG, 64, 128), lambda: (0, 0, 0)),
            pl.BlockSpec((_NSEG, 1), lambda: (0, 0)),
            pl.BlockSpec((_NSEG, 1), lambda: (0, 0)),
        ],
        out_specs=pl.BlockSpec((_NSEG, 512), lambda: (0, 0)),
        out_shape=jax.ShapeDtypeStruct((_NSEG, 512), jnp.float32),
    )(pa, pb, da, db, ma, mb)


def kernel(x, W1, b1, W2, b2, batch):
    n, d = x.shape
    h = W1.shape[1]
    nseg = _NSEG
    blk = 2000
    nbh = _HALF // blk

    batch_i = batch.astype(jnp.int32)
    batch3 = batch_i.reshape(n // blk, 1, blk)
    b1r = b1.reshape(1, h)

    lga, ma = _k1(x, W1, b1r, W2, batch3, nseg, blk, nbh, d, h, 0)
    lgb, mb = _k1(x, W1, b1r, W2, batch3, nseg, blk, nbh, d, h, nbh)
    fa, da = _k2(x, lga.reshape(_HALF), batch_i[:_HALF], ma.reshape(nseg), 0)
    fb, db = _k2(x, lgb.reshape(_HALF), batch_i[_HALF:], mb.reshape(nseg),
                 _HALF)
    return _k3(fa.reshape(_RG, _CG, _TR, 128), fb.reshape(_RG, _CG, _TR, 128),
               da, db, ma, mb)


# speedup vs baseline: 1.0001x; 1.0001x over previous
import functools

import jax
import jax.numpy as jnp
from jax import lax
from jax.experimental import pallas as pl
from jax.experimental.pallas import tpu as pltpu
from jax.experimental.pallas import tpu_sc as plsc

_NEG = float("-inf")

_N = 100000
_L = 16
_CH = 128
_NSEG = 512
_RG = 8
_CG = 4
_TR = 520


def _k1_body(x_ref, w1_ref, b1_ref, w2_ref, batch_ref, lg_ref, m_ref,
             rmax_ref, *, nseg, blk):
    i = pl.program_id(0)
    nb = pl.num_programs(0)

    @pl.when(i == 0)
    def _init():
        rmax_ref[...] = jnp.full((nseg, 1), _NEG, jnp.bfloat16)

    x = x_ref[...]
    h = jnp.dot(x, w1_ref[...], preferred_element_type=jnp.float32)
    h = h + b1_ref[...]
    h = h * jax.nn.sigmoid(h)
    lt_row = lax.dot_general(w2_ref[...], h, (((0,), (1,)), ((), ())),
                             preferred_element_type=jnp.float32)
    lg_ref[...] = lt_row.reshape(1, 1, blk)

    bt = batch_ref[0].astype(jnp.int16)
    seg = lax.broadcasted_iota(jnp.int16, (nseg, 1), 0)
    oh = bt == seg
    ltb = lt_row.astype(jnp.bfloat16)
    bmax = jnp.max(jnp.where(oh, ltb, jnp.bfloat16(_NEG)),
                   axis=1, keepdims=True)
    nm = jnp.maximum(rmax_ref[...], bmax)
    rmax_ref[...] = nm

    @pl.when(i == nb - 1)
    def _fin():
        m_ref[...] = nm.astype(jnp.float32)


def _k1(x, W1, b1r, w2r, batch3, nseg, blk, nb, d, h):
    return pl.pallas_call(
        functools.partial(_k1_body, nseg=nseg, blk=blk),
        grid=(nb,),
        in_specs=[
            pl.BlockSpec((blk, d), lambda i: (i, 0)),
            pl.BlockSpec((d, h), lambda i: (0, 0)),
            pl.BlockSpec((1, h), lambda i: (0, 0)),
            pl.BlockSpec((h, 1), lambda i: (0, 0)),
            pl.BlockSpec((1, 1, blk), lambda i: (i, 0, 0)),
        ],
        out_specs=[
            pl.BlockSpec((1, 1, blk), lambda i: (i, 0, 0)),
            pl.BlockSpec((nseg, 1), lambda i: (0, 0)),
        ],
        out_shape=[
            jax.ShapeDtypeStruct((nb, 1, blk), jnp.float32),
            jax.ShapeDtypeStruct((nseg, 1), jnp.float32),
        ],
        scratch_shapes=[pltpu.VMEM((nseg, 1), jnp.bfloat16)],
    )(x, W1, b1r, w2r, batch3)


def _k2_body(x_hbm, lg_hbm, b_hbm, m_hbm, feat_hbm, den_hbm,
             xv, lv, bv, mv, tab, dtab, sems):
    c = lax.axis_index("c")
    s = lax.axis_index("s")
    wid = c * 16 + s
    rw = wid // _CG
    cg = wid % _CG
    c0 = 97 * rw + jnp.minimum(rw, 5)
    cnt = 97 + (rw < 5).astype(jnp.int32) + (rw == 7).astype(jnp.int32)

    pltpu.sync_copy(m_hbm, mv)

    zero = jnp.zeros((_L,), jnp.float32)

    def zrow(i, carry):
        for k in range(128 // _L):
            tab[i, pl.ds(k * _L, _L)] = zero
        return carry

    lax.fori_loop(0, _TR, zrow, 0)

    def zdrow(i, carry):
        for k in range(128 // _L):
            dtab[i, pl.ds(k * _L, _L)] = zero
        return carry

    lax.fori_loop(0, 64, zdrow, 0)

    iota = lax.iota(jnp.int32, _L)
    lane0 = (iota == 0).astype(jnp.float32)
    col_base = cg * 128

    def chunk_base(ci):
        return jnp.minimum(ci * _CH, _N - _CH)

    def fire(ci, b):
        base = chunk_base(ci)
        pltpu.async_copy(
            x_hbm.at[pl.ds(base, _CH), pl.ds(col_base, 128)], xv.at[b],
            sems.at[b])
        pltpu.async_copy(lg_hbm.at[pl.ds(base, _CH)], lv.at[b], sems.at[b])
        pltpu.async_copy(b_hbm.at[pl.ds(base, _CH)], bv.at[b], sems.at[b])

    def drain(ci, b):
        base = chunk_base(ci)
        pltpu.make_async_copy(
            x_hbm.at[pl.ds(base, _CH), pl.ds(col_base, 128)], xv.at[b],
            sems.at[b]).wait()
        pltpu.make_async_copy(
            lg_hbm.at[pl.ds(base, _CH)], lv.at[b], sems.at[b]).wait()
        pltpu.make_async_copy(
            b_hbm.at[pl.ds(base, _CH)], bv.at[b], sems.at[b]).wait()

    def flush(tgt, acc, accd):
        for k in range(128 // _L):
            tab[tgt, pl.ds(k * _L, _L)] = tab[tgt, pl.ds(k * _L, _L)] + acc[k]
        plsc.addupdate_scatter(
            dtab,
            [jnp.full((_L,), lax.shift_right_logical(tgt, 3), jnp.int32),
             jnp.full((_L,), lax.bitwise_and(tgt, 7) * _L, jnp.int32) + iota],
            accd)

    fire(c0, 0)

    def chunk_body(i, carry):
        ci = c0 + i
        b = lax.rem(i, 2)

        @pl.when(i + 1 < cnt)
        def _():
            fire(ci + 1, 1 - b)

        drain(ci, b)
        glo = jnp.where(ci * _CH > _N - _CH, (_CH - 32) // _L, 0)

        def group_body(g, gc):
            acc = gc[:8]
            accd = gc[8]
            cur = gc[9]
            b16 = bv[b, pl.ds(g * _L, _L)]
            l16 = lv[b, pl.ds(g * _L, _L)]
            m16 = plsc.load_gather(mv, [b16])
            e16 = jnp.exp(l16 - m16)
            seg0 = b16[0]

            def fast(*op):
                facc = list(op[:8])
                faccd = op[8]
                fcur = op[9]

                @pl.when(seg0 != fcur)
                def _():
                    flush(jnp.maximum(fcur, 0), facc, faccd)

                keep = jnp.where(seg0 == fcur, 1.0, 0.0)
                facc = [a * keep for a in facc]
                faccd = faccd * keep + e16
                for r in range(_L):
                    e_b = e16[r]
                    for k in range(128 // _L):
                        facc[k] = facc[k] + e_b * xv[
                            b, g * _L + r, pl.ds(k * _L, _L)]
                return tuple(facc) + (faccd, seg0)

            def slow(*op):
                sacc = list(op[:8])
                saccd = op[8]
                scur = op[9]
                for r in range(_L):
                    seg = b16[r]
                    e_b = e16[r]

                    @pl.when(seg != scur)
                    def _(sacc=sacc, saccd=saccd, scur=scur):
                        flush(jnp.maximum(scur, 0), sacc, saccd)

                    keep = jnp.where(seg == scur, 1.0, 0.0)
                    for k in range(128 // _L):
                        sacc[k] = sacc[k] * keep + e_b * xv[
                            b, g * _L + r, pl.ds(k * _L, _L)]
                    saccd = saccd * keep + e_b * lane0
                    scur = seg
                return tuple(sacc) + (saccd, scur)

            return lax.cond(seg0 == b16[_L - 1], fast, slow, *gc)

        return lax.fori_loop(glo, _CH // _L, group_body, carry)

    carry0 = tuple(zero for _ in range(9)) + (jnp.int32(-1),)
    fc = lax.fori_loop(0, cnt, chunk_body, carry0)
    flush(jnp.maximum(fc[9], 0), list(fc[:8]), fc[8])

    pltpu.sync_copy(tab, feat_hbm.at[wid])
    pltpu.sync_copy(dtab, den_hbm.at[wid])


def _k2(x, lg, batch, m):
    mesh = plsc.VectorSubcoreMesh(core_axis_name="c", subcore_axis_name="s")
    f = pl.kernel(
        _k2_body,
        out_type=[
            jax.ShapeDtypeStruct((_RG * _CG, _TR, 128), jnp.float32),
            jax.ShapeDtypeStruct((_RG * _CG, 64, 128), jnp.float32),
        ],
        mesh=mesh,
        compiler_params=pltpu.CompilerParams(needs_layout_passes=False),
        scratch_types=[
            pltpu.VMEM((2, _CH, 128), jnp.float32),
            pltpu.VMEM((2, _CH), jnp.float32),
            pltpu.VMEM((2, _CH), jnp.int32),
            pltpu.VMEM((_NSEG,), jnp.float32),
            pltpu.VMEM((_TR, 128), jnp.float32),
            pltpu.VMEM((64, 128), jnp.float32),
            pltpu.SemaphoreType.DMA((2,)),
        ],
    )
    return f(x, lg, batch, m)


def _k3_body(p_ref, d_ref, out_ref):
    p = p_ref[...][:, :, :_NSEG, :]
    psum = jnp.sum(p, axis=0)
    feat = jnp.concatenate([psum[g] for g in range(_CG)], axis=1)
    dsum = jnp.sum(d_ref[...], axis=0)
    srow = lax.broadcasted_iota(jnp.int32, (_NSEG, 64), 0)
    rcol = lax.broadcasted_iota(jnp.int32, (_NSEG, 64), 1)
    sel = (rcol == lax.shift_right_logical(srow, 3)).astype(jnp.float32)
    g = jnp.dot(sel, dsum, preferred_element_type=jnp.float32)
    sc = lax.broadcasted_iota(jnp.int32, (_NSEG, 128), 0)
    cc = lax.broadcasted_iota(jnp.int32, (_NSEG, 128), 1)
    win = (lax.shift_right_logical(cc, 4) ==
           lax.bitwise_and(sc, 7)).astype(jnp.float32)
    den = jnp.sum(g * win, axis=1, keepdims=True) * 0.25
    out_ref[...] = feat / (den + 1e-16)


def _k3(p4, d4):
    return pl.pallas_call(
        _k3_body,
        in_specs=[
            pl.BlockSpec((_RG, _CG, _TR, 128), lambda: (0, 0, 0, 0)),
            pl.BlockSpec((_RG * _CG, 64, 128), lambda: (0, 0, 0)),
        ],
        out_specs=pl.BlockSpec((_NSEG, 512), lambda: (0, 0)),
        out_shape=jax.ShapeDtypeStruct((_NSEG, 512), jnp.float32),
    )(p4, d4)


def kernel(x, W1, b1, W2, b2, batch):
    n, d = x.shape
    h = W1.shape[1]
    nseg = _NSEG
    blk = 4000
    nb = n // blk

    batch_i = batch.astype(jnp.int32)
    batch3 = batch_i.reshape(nb, 1, blk)
    b1r = b1.reshape(1, h)

    lg3, m = _k1(x, W1, b1r, W2, batch3, nseg, blk, nb, d, h)
    feat, den = _k2(x, lg3.reshape(n), batch_i, m.reshape(nseg))
    return _k3(feat.reshape(_RG, _CG, _TR, 128), den)
